# baseline (speedup 1.0000x reference)
import jax
import jax.numpy as jnp
from jax.experimental import pallas as pl
from jax.experimental.pallas import tpu as pltpu

_N = 16 * 3 * 512 * 512
_ROWS = 12288
_COLS = 1024
_BLOCK_ROWS = 1024
_GRID = _ROWS // _BLOCK_ROWS


def _l1_mean_kernel(a_ref, b_ref, o_ref):
    i = pl.program_id(0)
    part = jnp.sum(jnp.abs(a_ref[...] - b_ref[...])) * (1.0 / _N)

    @pl.when(i == 0)
    def _init():
        o_ref[0, 0] = part

    @pl.when(i > 0)
    def _acc():
        o_ref[0, 0] += part


def kernel(inp, tar, boxes, texts):
    a = inp.reshape(_ROWS, _COLS)
    b = tar.reshape(_ROWS, _COLS)
    out = pl.pallas_call(
        _l1_mean_kernel,
        grid=(_GRID,),
        in_specs=[
            pl.BlockSpec((_BLOCK_ROWS, _COLS), lambda i: (i, 0)),
            pl.BlockSpec((_BLOCK_ROWS, _COLS), lambda i: (i, 0)),
        ],
        out_specs=pl.BlockSpec(
            (1, 1), lambda i: (0, 0), memory_space=pltpu.SMEM
        ),
        out_shape=jax.ShapeDtypeStruct((1, 1), jnp.float32),
    )(a, b)
    return out[0, 0]
